# Initial kernel scaffold; baseline (speedup 1.0000x reference)
import jax
import jax.numpy as jnp
from jax.experimental import pallas as pl


def kernel(x, gate_w, gate_b, w1, b1, w2, b2):
    raise NotImplementedError("write your pallas kernel here")


# speedup vs baseline: 3.6348x; 3.6348x over previous
import functools

import jax
import jax.numpy as jnp
from jax.experimental import pallas as pl
from jax.experimental.pallas import tpu as pltpu

NE = 8
TOPK = 2
C = 768
H = 3072
BLK = 256
NA = 2048 * TOPK
NB = NA // BLK + NE - 1
NPAD = NB * BLK


def _gate_body(x_ref, gw_ref, gb_ref, e_ref, w_ref):
    x = x_ref[...]
    logits = jax.lax.dot_general(
        x, gw_ref[...], (((1,), (1,)), ((), ())),
        preferred_element_type=jnp.float32)
    logits = logits + gb_ref[...]
    m = jnp.max(logits, axis=-1, keepdims=True)
    ex = jnp.exp(logits - m)
    p = ex / jnp.sum(ex, axis=-1, keepdims=True)
    iota = jax.lax.broadcasted_iota(jnp.int32, p.shape, 1)
    m0 = jnp.max(p, axis=-1, keepdims=True)
    i0 = jnp.min(jnp.where(p == m0, iota, NE), axis=-1, keepdims=True)
    p2 = jnp.where(iota == i0, -jnp.inf, p)
    m1 = jnp.max(p2, axis=-1, keepdims=True)
    i1 = jnp.min(jnp.where(p2 == m1, iota, NE), axis=-1, keepdims=True)
    s = m0 + m1
    e_ref[...] = jnp.concatenate([i0, i1], axis=1)
    w_ref[...] = jnp.concatenate([m0 / s, m1 / s], axis=1)


def _gelu(h):
    return h * 0.5 * (1.0 + jax.lax.erf(h * 0.7071067811865476))


def _ffn_body(be_ref, valid_ref, xs_ref, w1_ref, b1_ref, w2_ref, b2_ref,
              wgt_ref, ys_ref):
    j = pl.program_id(0)

    @pl.when(valid_ref[j] != 0)
    def _():
        xs = xs_ref[...]
        h = jax.lax.dot_general(
            xs, w1_ref[0], (((1,), (1,)), ((), ())),
            preferred_element_type=jnp.float32)
        h = _gelu(h + b1_ref[0])
        y = jax.lax.dot_general(
            h, w2_ref[0], (((1,), (1,)), ((), ())),
            preferred_element_type=jnp.float32)
        y = y + b2_ref[0]
        ys_ref[...] = y * wgt_ref[...]


def kernel(x, gate_w, gate_b, w1, b1, w2, b2):
    Bs, T, _ = x.shape
    xr = x.reshape(Bs * T, C)

    e_idx, wts = pl.pallas_call(
        _gate_body,
        out_shape=(
            jax.ShapeDtypeStruct((Bs * T, TOPK), jnp.int32),
            jax.ShapeDtypeStruct((Bs * T, TOPK), jnp.float32),
        ),
    )(xr, gate_w, gate_b.reshape(1, NE))

    ex = e_idx.reshape(-1)
    wv = wts.reshape(-1)
    oh = (ex[:, None] == jnp.arange(NE, dtype=jnp.int32)[None, :]).astype(jnp.int32)
    csum = jnp.cumsum(oh, axis=0)
    rank = jnp.sum((csum - oh) * oh, axis=1)
    g = csum[-1]
    nb = (g + BLK - 1) // BLK
    startpad = (jnp.cumsum(nb) - nb) * BLK
    pos = startpad[ex] + rank
    tok = jnp.arange(NA, dtype=jnp.int32) // TOPK
    sorted_tok = jnp.zeros((NPAD,), jnp.int32).at[pos].set(tok)
    wgtpad = jnp.zeros((NPAD,), jnp.float32).at[pos].set(wv)
    cnb = jnp.cumsum(nb)
    total = cnb[-1]
    jidx = jnp.arange(NB, dtype=jnp.int32)
    be_raw = jnp.sum((jidx[:, None] >= cnb[None, :]).astype(jnp.int32), axis=1)
    valid = (jidx < total).astype(jnp.int32)
    be_last = jnp.clip(be_raw, 0, NE - 1)[total - 1]
    be = jnp.where(valid == 1, be_raw, be_last).astype(jnp.int32)

    xs = jnp.take(xr, sorted_tok, axis=0)

    ys = pl.pallas_call(
        _ffn_body,
        grid_spec=pltpu.PrefetchScalarGridSpec(
            num_scalar_prefetch=2,
            grid=(NB,),
            in_specs=[
                pl.BlockSpec((BLK, C), lambda j, be, vd: (j, 0)),
                pl.BlockSpec((1, H, C), lambda j, be, vd: (be[j], 0, 0)),
                pl.BlockSpec((1, 1, H), lambda j, be, vd: (be[j], 0, 0)),
                pl.BlockSpec((1, C, H), lambda j, be, vd: (be[j], 0, 0)),
                pl.BlockSpec((1, 1, C), lambda j, be, vd: (be[j], 0, 0)),
                pl.BlockSpec((BLK, 1), lambda j, be, vd: (j, 0)),
            ],
            out_specs=pl.BlockSpec((BLK, C), lambda j, be, vd: (j, 0)),
        ),
        out_shape=jax.ShapeDtypeStruct((NPAD, C), jnp.float32),
        compiler_params=pltpu.CompilerParams(
            dimension_semantics=("arbitrary",),
        ),
    )(be, valid, xs, w1, b1.reshape(NE, 1, H), w2, b2.reshape(NE, 1, C),
      wgtpad.reshape(NPAD, 1))

    out = jnp.take(ys, pos[0::2], axis=0) + jnp.take(ys, pos[1::2], axis=0)
    return out.reshape(Bs, T, C)
